# Initial kernel scaffold; baseline (speedup 1.0000x reference)
import jax
import jax.numpy as jnp
from jax.experimental import pallas as pl


def kernel(input_ids, token_table, pos_table):
    raise NotImplementedError("write your pallas kernel here")


# speedup vs baseline: 1.1401x; 1.1401x over previous
import functools

import jax
import jax.numpy as jnp
from jax import lax
from jax.experimental import pallas as pl
from jax.experimental.pallas import tpu as pltpu
from jax.experimental.pallas import tpu_sc as plsc

_NUM_CORES = 2
_NUM_SUBCORES = 16
_NUM_WORKERS = _NUM_CORES * _NUM_SUBCORES
_LANES = 16

_CH = 2


def _make_kernel(B, S, D, V):
    assert (B * S) % _NUM_WORKERS == 0
    rows_per_w = B // _NUM_WORKERS
    assert rows_per_w % _CH == 0
    n_chunks = rows_per_w // _CH
    ch_rows = _CH * S
    flat_per_w = rows_per_w * S
    n_h = D // _LANES

    mesh = plsc.VectorSubcoreMesh(core_axis_name="c", subcore_axis_name="s")

    @functools.partial(
        pl.kernel,
        mesh=mesh,
        out_type=jax.ShapeDtypeStruct((B * S, D), jnp.float32),
        scratch_types=[
            pltpu.VMEM((S, D), jnp.float32),
            pltpu.VMEM((ch_rows,), jnp.int32),
            pltpu.VMEM((ch_rows, D), jnp.float32),
            pltpu.SemaphoreType.DMA,
        ],
        compiler_params=pltpu.CompilerParams(use_tc_tiling_on_sc=False),
    )
    def embed(ids_hbm, table_hbm, pos_hbm, out_hbm, pos_v, idx_v, rows_v, sem):
        wid = lax.axis_index("s") * _NUM_CORES + lax.axis_index("c")
        base = wid * flat_per_w

        pltpu.sync_copy(pos_hbm, pos_v)

        def chunk_body(g, _):
            off = base + g * ch_rows
            pltpu.sync_copy(ids_hbm.at[pl.ds(off, ch_rows)], idx_v)
            pltpu.async_copy(table_hbm.at[idx_v], rows_v, sem).wait()

            def add_pos(s, _):
                for rb in range(_CH):
                    for h in range(n_h):
                        sl = pl.ds(h * _LANES, _LANES)
                        rows_v[rb * S + s, sl] += pos_v[s, sl]
                return ()

            lax.fori_loop(0, S, add_pos, (), unroll=4)
            pltpu.sync_copy(rows_v, out_hbm.at[pl.ds(off, ch_rows)])
            return ()

        lax.fori_loop(0, n_chunks, chunk_body, ())

    return embed


def kernel(input_ids, token_table, pos_table):
    B, S = input_ids.shape
    V, D = token_table.shape
    ids_flat = input_ids.reshape(B * S).astype(jnp.int32)
    out = _make_kernel(B, S, D, V)(ids_flat, token_table, pos_table)
    return out.reshape(B, S, D)


# speedup vs baseline: 1.3683x; 1.2002x over previous
import functools

import jax
import jax.numpy as jnp
from jax import lax
from jax.experimental import pallas as pl
from jax.experimental.pallas import tpu as pltpu
from jax.experimental.pallas import tpu_sc as plsc

_NUM_CORES = 2
_NUM_SUBCORES = 16
_NUM_WORKERS = _NUM_CORES * _NUM_SUBCORES
_LANES = 16

_CH = 2


def _make_kernel(B, S, D, V):
    assert (B * S) % _NUM_WORKERS == 0
    rows_per_w = B // _NUM_WORKERS
    assert rows_per_w % _CH == 0
    n_chunks = rows_per_w // _CH
    assert n_chunks % 2 == 0 and n_chunks >= 4
    ch_rows = _CH * S
    flat_per_w = rows_per_w * S
    n_h = D // _LANES

    mesh = plsc.VectorSubcoreMesh(core_axis_name="c", subcore_axis_name="s")

    @functools.partial(
        pl.kernel,
        mesh=mesh,
        out_type=jax.ShapeDtypeStruct((B * S, D), jnp.float32),
        scratch_types=[
            pltpu.VMEM((S, D), jnp.float32),
            pltpu.VMEM((ch_rows,), jnp.int32),
            pltpu.VMEM((ch_rows,), jnp.int32),
            pltpu.VMEM((ch_rows, D), jnp.float32),
            pltpu.VMEM((ch_rows, D), jnp.float32),
            pltpu.SemaphoreType.DMA,
            pltpu.SemaphoreType.DMA,
            pltpu.SemaphoreType.DMA,
            pltpu.SemaphoreType.DMA,
            pltpu.SemaphoreType.DMA,
            pltpu.SemaphoreType.DMA,
        ],
        compiler_params=pltpu.CompilerParams(use_tc_tiling_on_sc=False),
    )
    def embed(ids_hbm, table_hbm, pos_hbm, out_hbm,
              pos_v, idx0, idx1, rows0, rows1,
              isem0, isem1, gsem0, gsem1, osem0, osem1):
        wid = lax.axis_index("s") * _NUM_CORES + lax.axis_index("c")
        base = wid * flat_per_w
        idx = (idx0, idx1)
        rows = (rows0, rows1)
        isem = (isem0, isem1)
        gsem = (gsem0, gsem1)
        osem = (osem0, osem1)

        def ids_src(g):
            return ids_hbm.at[pl.ds(base + g * ch_rows, ch_rows)]

        def out_dst(g):
            return out_hbm.at[pl.ds(base + g * ch_rows, ch_rows)]

        pltpu.sync_copy(pos_hbm, pos_v)

        pltpu.async_copy(ids_src(0), idx[0], isem[0])
        pltpu.async_copy(ids_src(1), idx[1], isem[1])
        pltpu.make_async_copy(ids_src(0), idx[0], isem[0]).wait()
        pltpu.async_copy(table_hbm.at[idx[0]], rows[0], gsem[0])

        @pl.loop(0, n_chunks, step=2)
        def chunk_loop(g0):
            for b in range(2):
                g = g0 + b
                nb = 1 - b
                pltpu.make_async_copy(table_hbm.at[idx[b]], rows[b],
                                      gsem[b]).wait()

                @pl.when(g + 2 < n_chunks)
                def _():
                    pltpu.async_copy(ids_src(g + 2), idx[b], isem[b])

                @pl.when(g + 1 < n_chunks)
                def _():
                    pltpu.make_async_copy(ids_src(g + 1), idx[nb],
                                          isem[nb]).wait()

                    @pl.when(g >= 1)
                    def _():
                        pltpu.make_async_copy(rows[nb], out_dst(g - 1),
                                              osem[nb]).wait()

                    pltpu.async_copy(table_hbm.at[idx[nb]], rows[nb],
                                     gsem[nb])

                @pl.loop(0, S, unroll=4)
                def add_pos(s):
                    for h in range(n_h):
                        sl = pl.ds(h * _LANES, _LANES)
                        pv = pos_v[s, sl]
                        for rb in range(_CH):
                            rows[b][rb * S + s, sl] += pv

                pltpu.async_copy(rows[b], out_dst(g), osem[b])

        pltpu.make_async_copy(rows[0], out_dst(n_chunks - 2), osem[0]).wait()
        pltpu.make_async_copy(rows[1], out_dst(n_chunks - 1), osem[1]).wait()

    return embed


def kernel(input_ids, token_table, pos_table):
    B, S = input_ids.shape
    V, D = token_table.shape
    ids_flat = input_ids.reshape(B * S).astype(jnp.int32)
    out = _make_kernel(B, S, D, V)(ids_flat, token_table, pos_table)
    return out.reshape(B, S, D)
